# Initial kernel scaffold; baseline (speedup 1.0000x reference)
import functools

import jax
import jax.numpy as jnp
from jax import lax
from jax.experimental import pallas as pl
from jax.experimental.pallas import tpu as pltpu
from jax.experimental.pallas import tpu_sc as plsc

TOKEN_BLOCK = 256
CHUNK = 128


def _argmin_body(z_ref, et_ref, idx_ref, idx_out_ref):
    zb = z_ref[...]
    et = et_ref[...]
    esq = jnp.sum(et * et, axis=0, keepdims=True)
    dot = jax.lax.dot_general(
        zb, et, (((1,), (0,)), ((), ())),
        preferred_element_type=jnp.float32)
    d = esq - 2.0 * dot
    m = jnp.min(d, axis=1, keepdims=True)
    iota = lax.broadcasted_iota(jnp.int32, d.shape, 1)
    big = jnp.int32(2 ** 30)
    idx = jnp.min(jnp.where(d == m, iota, big), axis=1)
    idx_ref[...] = idx
    idx_out_ref[...] = idx + 32000


def _nearest_codes(z_flat, e_t):
    n_tok, e_dim = z_flat.shape
    n_e = e_t.shape[1]
    grid = n_tok // TOKEN_BLOCK
    return pl.pallas_call(
        _argmin_body,
        grid=(grid,),
        in_specs=[
            pl.BlockSpec((TOKEN_BLOCK, e_dim), lambda i: (i, 0)),
            pl.BlockSpec((e_dim, n_e), lambda i: (0, 0)),
        ],
        out_specs=[
            pl.BlockSpec((TOKEN_BLOCK,), lambda i: (i,)),
            pl.BlockSpec((TOKEN_BLOCK,), lambda i: (i,)),
        ],
        out_shape=[
            jax.ShapeDtypeStruct((n_tok,), jnp.int32),
            jax.ShapeDtypeStruct((n_tok,), jnp.int32),
        ],
    )(z_flat, e_t)


def _gather_rows(table, idx):
    n_tok = idx.shape[0]
    e_dim = table.shape[1]
    info = plsc.get_sparse_core_info()
    nw = info.num_cores * info.num_subcores
    b_per_w = n_tok // nw
    n_chunks = b_per_w // CHUNK
    mesh = plsc.VectorSubcoreMesh(core_axis_name="c", subcore_axis_name="s")

    @functools.partial(
        pl.kernel, mesh=mesh,
        out_type=jax.ShapeDtypeStruct((n_tok, e_dim), jnp.float32),
        scratch_types=[
            pltpu.VMEM((n_chunks, CHUNK), jnp.int32),
            pltpu.VMEM((b_per_w, e_dim), jnp.float32),
            pltpu.SemaphoreType.DMA,
        ],
    )
    def gather(table_hbm, idx_hbm, out_hbm, idx_v, rows_v, sem):
        wid = lax.axis_index("s") * info.num_cores + lax.axis_index("c")
        base = wid * b_per_w
        pltpu.sync_copy(idx_hbm.at[pl.ds(base, b_per_w)],
                        idx_v.reshape(b_per_w))
        copies = []
        for j in range(n_chunks):
            copies.append(pltpu.async_copy(
                table_hbm.at[idx_v.at[j]],
                rows_v.at[pl.ds(j * CHUNK, CHUNK)], sem))
        for c in copies:
            c.wait()
        pltpu.sync_copy(rows_v, out_hbm.at[pl.ds(base, b_per_w)])

    return gather(table, idx)


def kernel(z, embedding_weight):
    bz = z.shape[0]
    e_dim = embedding_weight.shape[-1]
    z_flat = z.reshape(-1, e_dim)
    e_t = embedding_weight.T
    idx_raw, idx_out = _nearest_codes(z_flat, e_t)
    z_q = _gather_rows(embedding_weight, idx_raw)
    return z_q.reshape(z.shape), idx_out.reshape(bz, -1)


# speedup vs baseline: 12.3949x; 12.3949x over previous
import functools

import jax
import jax.numpy as jnp
from jax import lax
from jax.experimental import pallas as pl
from jax.experimental.pallas import tpu as pltpu
from jax.experimental.pallas import tpu_sc as plsc

TOKEN_BLOCK = 256
CHUNK = 128


def _argmin_body(z_ref, et_ref, zsq_ref, esq_ref, idx_ref, idx_out_ref):
    zb = z_ref[...]
    et = et_ref[...]
    dot = jax.lax.dot_general(
        zb, et, (((1,), (0,)), ((), ())),
        preferred_element_type=jnp.float32)
    d = (zsq_ref[...] + esq_ref[...]) - 2.0 * dot
    m = jnp.min(d, axis=1, keepdims=True)
    iota = lax.broadcasted_iota(jnp.int32, d.shape, 1)
    big = jnp.int32(2 ** 30)
    idx = jnp.min(jnp.where(d == m, iota, big), axis=1)
    idx_ref[...] = idx
    idx_out_ref[...] = idx + 32000


def _nearest_codes(z_flat, e_t, zsq, esq):
    n_tok, e_dim = z_flat.shape
    n_e = e_t.shape[1]
    grid = n_tok // TOKEN_BLOCK
    return pl.pallas_call(
        _argmin_body,
        grid=(grid,),
        in_specs=[
            pl.BlockSpec((TOKEN_BLOCK, e_dim), lambda i: (i, 0)),
            pl.BlockSpec((e_dim, n_e), lambda i: (0, 0)),
            pl.BlockSpec((TOKEN_BLOCK, 1), lambda i: (i, 0)),
            pl.BlockSpec((1, n_e), lambda i: (0, 0)),
        ],
        out_specs=[
            pl.BlockSpec((TOKEN_BLOCK,), lambda i: (i,)),
            pl.BlockSpec((TOKEN_BLOCK,), lambda i: (i,)),
        ],
        out_shape=[
            jax.ShapeDtypeStruct((n_tok,), jnp.int32),
            jax.ShapeDtypeStruct((n_tok,), jnp.int32),
        ],
    )(z_flat, e_t, zsq, esq)


def _gather_rows(table, idx):
    n_tok = idx.shape[0]
    e_dim = table.shape[1]
    info = plsc.get_sparse_core_info()
    nw = info.num_cores * info.num_subcores
    b_per_w = n_tok // nw
    n_chunks = b_per_w // CHUNK
    mesh = plsc.VectorSubcoreMesh(core_axis_name="c", subcore_axis_name="s")

    @functools.partial(
        pl.kernel, mesh=mesh,
        out_type=jax.ShapeDtypeStruct((n_tok, e_dim), jnp.float32),
        compiler_params=pltpu.CompilerParams(use_tc_tiling_on_sc=False),
        scratch_types=[
            pltpu.VMEM((n_chunks, CHUNK), jnp.int32),
            pltpu.VMEM((b_per_w, e_dim), jnp.float32),
            pltpu.SemaphoreType.DMA,
        ],
    )
    def gather(table_hbm, idx_hbm, out_hbm, idx_v, rows_v, sem):
        wid = lax.axis_index("s") * info.num_cores + lax.axis_index("c")
        base = wid * b_per_w
        for j in range(n_chunks):
            pltpu.sync_copy(idx_hbm.at[pl.ds(base + j * CHUNK, CHUNK)],
                            idx_v.at[j])
        copies = []
        for j in range(n_chunks):
            copies.append(pltpu.async_copy(
                table_hbm.at[idx_v.at[j]],
                rows_v.at[pl.ds(j * CHUNK, CHUNK)], sem))
        for c in copies:
            c.wait()
        pltpu.sync_copy(rows_v, out_hbm.at[pl.ds(base, b_per_w)])

    return gather(table, idx)


def kernel(z, embedding_weight):
    bz = z.shape[0]
    e_dim = embedding_weight.shape[-1]
    z_flat = z.reshape(-1, e_dim)
    e_t = embedding_weight.T
    zsq = jnp.sum(z_flat ** 2, axis=1, keepdims=True)
    esq = jnp.sum(embedding_weight ** 2, axis=1)[None, :]
    idx_raw, idx_out = _nearest_codes(z_flat, e_t, zsq, esq)
    z_q = _gather_rows(embedding_weight, idx_raw)
    return z_q.reshape(z.shape), idx_out.reshape(bz, -1)
